# baseline (speedup 1.0000x reference)
import functools

import jax
import jax.numpy as jnp
from jax import lax
from jax.experimental import pallas as pl
from jax.experimental.pallas import tpu as pltpu
from jax.experimental.pallas import tpu_sc as plsc

N = 10000
E = 320000
D_IN = 128
D_HID = 128
D_HALF = D_HID // 2
D_CLS = 40
D_CLS_PAD = 48

NC = 2
NS = 16
NW = NC * NS
RPT = N // NS
CH = 125
KSEG = 80
NRING = 8
GAH = 7

_mesh = lambda: plsc.VectorSubcoreMesh(
    core_axis_name="c", subcore_axis_name="s", num_cores=NC, num_subcores=NS)


def _deg_body(dst_hbm, out_hbm, dst_v, deg_v):
    c = lax.axis_index("c")
    s = lax.axis_index("s")
    wid = s * NC + c
    et = E // NW
    pltpu.sync_copy(dst_hbm.at[pl.ds(wid * et, et)], dst_v)

    zeros16 = jnp.zeros((16,), jnp.float32)

    def zero_step(i, _):
        deg_v[pl.ds(i * 16, 16)] = zeros16
        return 0

    lax.fori_loop(0, N // 16, zero_step, 0)

    ones16 = jnp.ones((16,), jnp.float32)

    def acc_step(i, _):
        idx = dst_v[pl.ds(i * 16, 16)]
        plsc.addupdate_scatter(deg_v, [idx], ones16)
        return 0

    lax.fori_loop(0, E // NW // 16, acc_step, 0)
    pltpu.sync_copy(deg_v, out_hbm.at[wid])


def _deg_partials(dst):
    return pl.kernel(
        _deg_body,
        out_type=jax.ShapeDtypeStruct((NW, N), jnp.float32),
        mesh=_mesh(),
        scratch_types=[
            pltpu.VMEM((E // NW,), jnp.int32),
            pltpu.VMEM((N,), jnp.float32),
        ],
        compiler_params=pltpu.CompilerParams(needs_layout_passes=False),
    )(dst)


def _agg_body(d, nseg, l1, xs_hbm, src_hbm, dst_hbm, zrows_hbm, out_hbm,
              src_v, dst_v,
              b0, b1, b2, b3, b4, b5, b6, b7, acc_sh,
              g0, g1, g2, g3, g4, g5, g6, g7,
              t0, t1, t2, t3, t4, t5, t6, t7):
    bufs = (b0, b1, b2, b3, b4, b5, b6, b7)
    g = (g0, g1, g2, g3, g4, g5, g6, g7)
    t = (t0, t1, t2, t3, t4, t5, t6, t7)
    c = lax.axis_index("c")
    s = lax.axis_index("s")
    wid = s * NC + c
    pltpu.sync_copy(zrows_hbm, acc_sh.at[pl.ds(s * RPT, RPT)])
    plsc.subcore_barrier()

    for seg in range(nseg):
        if l1:
            pltpu.sync_copy(src_hbm.at[c, s, seg], src_v)
            pltpu.sync_copy(dst_hbm.at[s, seg], dst_v)
        else:
            pltpu.sync_copy(src_hbm.at[wid, seg], src_v)
            pltpu.sync_copy(dst_hbm.at[wid, seg], dst_v)

        for b in range(GAH):
            pltpu.async_copy(xs_hbm.at[src_v.at[b]], bufs[b], g[b])

        def round_(j, _):
            for b in range(NRING):
                k = NRING * j + b
                pltpu.make_async_copy(
                    xs_hbm.at[src_v.at[k]], bufs[b], g[b]).wait()
                pltpu.async_copy(bufs[b], acc_sh.at[dst_v.at[k]], t[b],
                                 add=True)
                bx = (b + GAH) % NRING
                if b < NRING - GAH:
                    @pl.when(j > 0)
                    def _():
                        pltpu.make_async_copy(
                            bufs[bx], acc_sh.at[dst_v.at[k - (NRING - GAH)]],
                            t[bx]).wait()
                    pltpu.async_copy(
                        xs_hbm.at[src_v.at[k + GAH]], bufs[bx], g[bx])
                else:
                    pltpu.make_async_copy(
                        bufs[bx], acc_sh.at[dst_v.at[k - (NRING - GAH)]],
                        t[bx]).wait()

                    @pl.when(j < KSEG // NRING - 1)
                    def _():
                        pltpu.async_copy(
                            xs_hbm.at[src_v.at[k + GAH]], bufs[bx], g[bx])
            return 0

        lax.fori_loop(0, KSEG // NRING, round_, 0)
        for b in range(GAH, NRING):
            pltpu.make_async_copy(
                bufs[b], acc_sh.at[dst_v.at[KSEG - NRING + b]], t[b]).wait()

    plsc.subcore_barrier()
    pltpu.sync_copy(acc_sh.at[pl.ds(s * RPT, RPT)],
                    out_hbm.at[c, pl.ds(s * RPT, RPT)])


def _edge_aggregate(xs, src_idx, dst_idx, d, nseg, l1):
    zrows = jnp.zeros((RPT, d), jnp.float32)
    return pl.kernel(
        functools.partial(_agg_body, d, nseg, l1),
        out_type=jax.ShapeDtypeStruct((NC, N, d), jnp.float32),
        mesh=_mesh(),
        scratch_types=(
            [pltpu.VMEM((KSEG, CH), jnp.int32),
             pltpu.VMEM((KSEG, CH), jnp.int32)]
            + [pltpu.VMEM((CH, d), jnp.float32) for _ in range(NRING)]
            + [pltpu.VMEM_SHARED((N, d), jnp.float32)]
            + [pltpu.SemaphoreType.DMA for _ in range(2 * NRING)]
        ),
        compiler_params=pltpu.CompilerParams(
            needs_layout_passes=False, use_tc_tiling_on_sc=False),
    )(xs, src_idx, dst_idx, zrows)


_BR = 1000


def _scale_mm_body(degp_ref, x_ref, w_ref, xs_ref, dinv_ref):
    ones_col = jnp.ones((NW, 1), jnp.float32)
    deg = lax.dot_general(degp_ref[...], ones_col, (((0,), (0,)), ((), ())),
                          preferred_element_type=jnp.float32) + 1.0
    dinv = lax.rsqrt(deg)
    xw = jnp.dot(x_ref[...], w_ref[...], preferred_element_type=jnp.float32)
    xs = dinv * xw
    xs_ref[0] = xs[:, :D_HALF]
    xs_ref[1] = xs[:, D_HALF:]
    dinv_ref[...] = dinv


def _scale_mm(degp, x, w1):
    return pl.pallas_call(
        _scale_mm_body,
        out_shape=[
            jax.ShapeDtypeStruct((NC, N, D_HALF), jnp.float32),
            jax.ShapeDtypeStruct((N, 1), jnp.float32),
        ],
    )(degp, x, w1)


def _mid_body(agg_ref, xs_ref, dinv_ref, b1_ref, w2_ref, hs_ref):
    agg = jnp.concatenate([agg_ref[0], agg_ref[1]], axis=1)
    xs = jnp.concatenate([xs_ref[0], xs_ref[1]], axis=1)
    dinv = dinv_ref[...]
    h = jnp.maximum(dinv * (agg + xs) + b1_ref[...], 0.0)
    hw = jnp.dot(h, w2_ref[...], preferred_element_type=jnp.float32)
    hs_ref[...] = dinv * hw


def _mid(agg, xs_fs, dinv, b1, w2p):
    return pl.pallas_call(
        _mid_body,
        grid=(N // _BR,),
        in_specs=[
            pl.BlockSpec((NC, _BR, D_HALF), lambda i: (0, i, 0)),
            pl.BlockSpec((NC, _BR, D_HALF), lambda i: (0, i, 0)),
            pl.BlockSpec((_BR, 1), lambda i: (i, 0)),
            pl.BlockSpec((1, D_HID), lambda i: (0, 0)),
            pl.BlockSpec((D_HID, D_CLS_PAD), lambda i: (0, 0)),
        ],
        out_specs=pl.BlockSpec((_BR, D_CLS_PAD), lambda i: (i, 0)),
        out_shape=jax.ShapeDtypeStruct((N, D_CLS_PAD), jnp.float32),
    )(agg, xs_fs, dinv, b1, w2p)


def _out_body(agg_ref, hs_ref, dinv_ref, b2_ref, o_ref):
    agg = agg_ref[0] + agg_ref[1]
    o = dinv_ref[...] * (agg + hs_ref[...]) + b2_ref[...]
    col = lax.broadcasted_iota(jnp.int32, (_BR, D_CLS_PAD), 1)
    mask = col < D_CLS
    m = jnp.max(jnp.where(mask, o, -jnp.inf), axis=1, keepdims=True)
    e = jnp.where(mask, jnp.exp(o - m), 0.0)
    ssum = jnp.sum(e, axis=1, keepdims=True)
    o_ref[...] = (o - m - jnp.log(ssum))[:, :D_CLS]


def _final(agg, hs, dinv, b2p):
    return pl.pallas_call(
        _out_body,
        grid=(N // _BR,),
        in_specs=[
            pl.BlockSpec((NC, _BR, D_CLS_PAD), lambda i: (0, i, 0)),
            pl.BlockSpec((_BR, D_CLS_PAD), lambda i: (i, 0)),
            pl.BlockSpec((_BR, 1), lambda i: (i, 0)),
            pl.BlockSpec((1, D_CLS_PAD), lambda i: (0, 0)),
        ],
        out_specs=pl.BlockSpec((_BR, D_CLS), lambda i: (i, 0)),
        out_shape=jax.ShapeDtypeStruct((N, D_CLS), jnp.float32),
    )(agg, hs, dinv, b2p)


def kernel(x, edge_index, W1, b1, W2, b2):
    src = edge_index[0].astype(jnp.int32)
    dst = edge_index[1].astype(jnp.int32)
    src_l1 = src.reshape(1, NS, 2, KSEG, CH) + (
        jnp.arange(NC, dtype=jnp.int32) * N).reshape(NC, 1, 1, 1, 1)
    dst_l1 = dst.reshape(NS, 2, KSEG, CH)
    src_l2 = src.reshape(NW, 1, KSEG, CH)
    dst_l2 = dst.reshape(NW, 1, KSEG, CH)

    deg_parts = _deg_partials(dst)

    xs_fs, dinv = _scale_mm(deg_parts, x.astype(jnp.float32), W1)
    xs2n = xs_fs.reshape(NC * N, D_HALF)
    agg1 = _edge_aggregate(xs2n, src_l1, dst_l1, D_HALF, 2, True)

    w2p = jnp.pad(W2, ((0, 0), (0, D_CLS_PAD - D_CLS)))
    b1r = b1.reshape(1, D_HID)
    hs = _mid(agg1, xs_fs, dinv, b1r, w2p)

    agg2 = _edge_aggregate(hs, src_l2, dst_l2, D_CLS_PAD, 1, False)
    b2p = jnp.pad(b2, (0, D_CLS_PAD - D_CLS)).reshape(1, D_CLS_PAD)
    return _final(agg2, hs, dinv, b2p)


# speedup vs baseline: 1.0023x; 1.0023x over previous
import functools

import jax
import jax.numpy as jnp
from jax import lax
from jax.experimental import pallas as pl
from jax.experimental.pallas import tpu as pltpu
from jax.experimental.pallas import tpu_sc as plsc

N = 10000
E = 320000
D_IN = 128
D_HID = 128
D_HALF = D_HID // 2
D_CLS = 40
D_CLS_PAD = 48

NC = 2
NS = 16
NW = NC * NS
RPT = N // NS
CH = 125
KSEG = 80
NRING = 8
GAH = 6

_mesh = lambda: plsc.VectorSubcoreMesh(
    core_axis_name="c", subcore_axis_name="s", num_cores=NC, num_subcores=NS)


def _deg_body(dst_hbm, out_hbm, dst_v, deg_v):
    c = lax.axis_index("c")
    s = lax.axis_index("s")
    wid = s * NC + c
    et = E // NW
    pltpu.sync_copy(dst_hbm.at[pl.ds(wid * et, et)], dst_v)

    zeros16 = jnp.zeros((16,), jnp.float32)

    def zero_step(i, _):
        deg_v[pl.ds(i * 16, 16)] = zeros16
        return 0

    lax.fori_loop(0, N // 16, zero_step, 0)

    ones16 = jnp.ones((16,), jnp.float32)

    def acc_step(i, _):
        idx = dst_v[pl.ds(i * 16, 16)]
        plsc.addupdate_scatter(deg_v, [idx], ones16)
        return 0

    lax.fori_loop(0, E // NW // 16, acc_step, 0)
    pltpu.sync_copy(deg_v, out_hbm.at[wid])


def _deg_partials(dst):
    return pl.kernel(
        _deg_body,
        out_type=jax.ShapeDtypeStruct((NW, N), jnp.float32),
        mesh=_mesh(),
        scratch_types=[
            pltpu.VMEM((E // NW,), jnp.int32),
            pltpu.VMEM((N,), jnp.float32),
        ],
        compiler_params=pltpu.CompilerParams(needs_layout_passes=False),
    )(dst)


def _agg_body(d, nseg, l1, xs_hbm, src_hbm, dst_hbm, zrows_hbm, out_hbm,
              src_v, dst_v,
              b0, b1, b2, b3, b4, b5, b6, b7, acc_sh,
              g0, g1, g2, g3, g4, g5, g6, g7,
              t0, t1, t2, t3, t4, t5, t6, t7):
    bufs = (b0, b1, b2, b3, b4, b5, b6, b7)
    g = (g0, g1, g2, g3, g4, g5, g6, g7)
    t = (t0, t1, t2, t3, t4, t5, t6, t7)
    c = lax.axis_index("c")
    s = lax.axis_index("s")
    wid = s * NC + c
    pltpu.sync_copy(zrows_hbm, acc_sh.at[pl.ds(s * RPT, RPT)])
    plsc.subcore_barrier()

    for seg in range(nseg):
        if l1:
            pltpu.sync_copy(src_hbm.at[c, s, seg], src_v)
            pltpu.sync_copy(dst_hbm.at[s, seg], dst_v)
        else:
            pltpu.sync_copy(src_hbm.at[wid, seg], src_v)
            pltpu.sync_copy(dst_hbm.at[wid, seg], dst_v)

        for b in range(GAH):
            pltpu.async_copy(xs_hbm.at[src_v.at[b]], bufs[b], g[b])

        def round_(j, _):
            for b in range(NRING):
                k = NRING * j + b
                pltpu.make_async_copy(
                    xs_hbm.at[src_v.at[k]], bufs[b], g[b]).wait()
                pltpu.async_copy(bufs[b], acc_sh.at[dst_v.at[k]], t[b],
                                 add=True)
                bx = (b + GAH) % NRING
                if b < NRING - GAH:
                    @pl.when(j > 0)
                    def _():
                        pltpu.make_async_copy(
                            bufs[bx], acc_sh.at[dst_v.at[k - (NRING - GAH)]],
                            t[bx]).wait()
                    pltpu.async_copy(
                        xs_hbm.at[src_v.at[k + GAH]], bufs[bx], g[bx])
                else:
                    pltpu.make_async_copy(
                        bufs[bx], acc_sh.at[dst_v.at[k - (NRING - GAH)]],
                        t[bx]).wait()

                    @pl.when(j < KSEG // NRING - 1)
                    def _():
                        pltpu.async_copy(
                            xs_hbm.at[src_v.at[k + GAH]], bufs[bx], g[bx])
            return 0

        lax.fori_loop(0, KSEG // NRING, round_, 0)
        for b in range(GAH, NRING):
            pltpu.make_async_copy(
                bufs[b], acc_sh.at[dst_v.at[KSEG - NRING + b]], t[b]).wait()

    plsc.subcore_barrier()
    pltpu.sync_copy(acc_sh.at[pl.ds(s * RPT, RPT)],
                    out_hbm.at[c, pl.ds(s * RPT, RPT)])


def _edge_aggregate(xs, src_idx, dst_idx, d, nseg, l1):
    zrows = jnp.zeros((RPT, d), jnp.float32)
    return pl.kernel(
        functools.partial(_agg_body, d, nseg, l1),
        out_type=jax.ShapeDtypeStruct((NC, N, d), jnp.float32),
        mesh=_mesh(),
        scratch_types=(
            [pltpu.VMEM((KSEG, CH), jnp.int32),
             pltpu.VMEM((KSEG, CH), jnp.int32)]
            + [pltpu.VMEM((CH, d), jnp.float32) for _ in range(NRING)]
            + [pltpu.VMEM_SHARED((N, d), jnp.float32)]
            + [pltpu.SemaphoreType.DMA for _ in range(2 * NRING)]
        ),
        compiler_params=pltpu.CompilerParams(
            needs_layout_passes=False, use_tc_tiling_on_sc=False),
    )(xs, src_idx, dst_idx, zrows)


_BR = 1000


def _scale_mm_body(degp_ref, x_ref, w_ref, xs_ref, dinv_ref):
    ones_col = jnp.ones((NW, 1), jnp.float32)
    deg = lax.dot_general(degp_ref[...], ones_col, (((0,), (0,)), ((), ())),
                          preferred_element_type=jnp.float32) + 1.0
    dinv = lax.rsqrt(deg)
    xw = jnp.dot(x_ref[...], w_ref[...], preferred_element_type=jnp.float32)
    xs = dinv * xw
    xs_ref[0] = xs[:, :D_HALF]
    xs_ref[1] = xs[:, D_HALF:]
    dinv_ref[...] = dinv


def _scale_mm(degp, x, w1):
    return pl.pallas_call(
        _scale_mm_body,
        out_shape=[
            jax.ShapeDtypeStruct((NC, N, D_HALF), jnp.float32),
            jax.ShapeDtypeStruct((N, 1), jnp.float32),
        ],
    )(degp, x, w1)


def _mid_body(agg_ref, xs_ref, dinv_ref, b1_ref, w2_ref, hs_ref):
    agg = jnp.concatenate([agg_ref[0], agg_ref[1]], axis=1)
    xs = jnp.concatenate([xs_ref[0], xs_ref[1]], axis=1)
    dinv = dinv_ref[...]
    h = jnp.maximum(dinv * (agg + xs) + b1_ref[...], 0.0)
    hw = jnp.dot(h, w2_ref[...], preferred_element_type=jnp.float32)
    hs_ref[...] = dinv * hw


def _mid(agg, xs_fs, dinv, b1, w2p):
    return pl.pallas_call(
        _mid_body,
        grid=(N // _BR,),
        in_specs=[
            pl.BlockSpec((NC, _BR, D_HALF), lambda i: (0, i, 0)),
            pl.BlockSpec((NC, _BR, D_HALF), lambda i: (0, i, 0)),
            pl.BlockSpec((_BR, 1), lambda i: (i, 0)),
            pl.BlockSpec((1, D_HID), lambda i: (0, 0)),
            pl.BlockSpec((D_HID, D_CLS_PAD), lambda i: (0, 0)),
        ],
        out_specs=pl.BlockSpec((_BR, D_CLS_PAD), lambda i: (i, 0)),
        out_shape=jax.ShapeDtypeStruct((N, D_CLS_PAD), jnp.float32),
    )(agg, xs_fs, dinv, b1, w2p)


def _out_body(agg_ref, hs_ref, dinv_ref, b2_ref, o_ref):
    agg = agg_ref[0] + agg_ref[1]
    o = dinv_ref[...] * (agg + hs_ref[...]) + b2_ref[...]
    col = lax.broadcasted_iota(jnp.int32, (_BR, D_CLS_PAD), 1)
    mask = col < D_CLS
    m = jnp.max(jnp.where(mask, o, -jnp.inf), axis=1, keepdims=True)
    e = jnp.where(mask, jnp.exp(o - m), 0.0)
    ssum = jnp.sum(e, axis=1, keepdims=True)
    o_ref[...] = (o - m - jnp.log(ssum))[:, :D_CLS]


def _final(agg, hs, dinv, b2p):
    return pl.pallas_call(
        _out_body,
        grid=(N // _BR,),
        in_specs=[
            pl.BlockSpec((NC, _BR, D_CLS_PAD), lambda i: (0, i, 0)),
            pl.BlockSpec((_BR, D_CLS_PAD), lambda i: (i, 0)),
            pl.BlockSpec((_BR, 1), lambda i: (i, 0)),
            pl.BlockSpec((1, D_CLS_PAD), lambda i: (0, 0)),
        ],
        out_specs=pl.BlockSpec((_BR, D_CLS), lambda i: (i, 0)),
        out_shape=jax.ShapeDtypeStruct((N, D_CLS), jnp.float32),
    )(agg, hs, dinv, b2p)


def kernel(x, edge_index, W1, b1, W2, b2):
    src = edge_index[0].astype(jnp.int32)
    dst = edge_index[1].astype(jnp.int32)
    src_l1 = src.reshape(1, NS, 2, KSEG, CH) + (
        jnp.arange(NC, dtype=jnp.int32) * N).reshape(NC, 1, 1, 1, 1)
    dst_l1 = dst.reshape(NS, 2, KSEG, CH)
    src_l2 = src.reshape(NW, 1, KSEG, CH)
    dst_l2 = dst.reshape(NW, 1, KSEG, CH)

    deg_parts = _deg_partials(dst)

    xs_fs, dinv = _scale_mm(deg_parts, x.astype(jnp.float32), W1)
    xs2n = xs_fs.reshape(NC * N, D_HALF)
    agg1 = _edge_aggregate(xs2n, src_l1, dst_l1, D_HALF, 2, True)

    w2p = jnp.pad(W2, ((0, 0), (0, D_CLS_PAD - D_CLS)))
    b1r = b1.reshape(1, D_HID)
    hs = _mid(agg1, xs_fs, dinv, b1r, w2p)

    agg2 = _edge_aggregate(hs, src_l2, dst_l2, D_CLS_PAD, 1, False)
    b2p = jnp.pad(b2, (0, D_CLS_PAD - D_CLS)).reshape(1, D_CLS_PAD)
    return _final(agg2, hs, dinv, b2p)
